# Initial kernel scaffold; baseline (speedup 1.0000x reference)
import jax
import jax.numpy as jnp
from jax.experimental import pallas as pl


def kernel(x_position_info, pos_post_emb, pos_para_emb, dist_post_emb, dist_para_emb):
    raise NotImplementedError("write your pallas kernel here")


# speedup vs baseline: 6.0063x; 6.0063x over previous
import jax
import jax.numpy as jnp
from jax import lax
from jax.experimental import pallas as pl
from jax.experimental.pallas import tpu as pltpu
from jax.experimental.pallas import tpu_sc as plsc

B = 1024
S = 50
HALF = 25
NW = 32
NB_W = B // NW
LANES = 16
REL_ROW = S * 32
REL_HB = HALF * REL_ROW
POS_ROW = S * 32
J_CHUNKS = (0, 16, 32, 34)


def _splat(v):
    return jnp.full((LANES,), v, jnp.int32)


def _body(x_hbm, ppost_hbm, ppara_hbm, dpost_hbm, dpara_hbm,
          pos_hbm, rel_hbm,
          x_v, ppost_v, ppara_v, dpost_v, dpara_v,
          rel_b0, rel_b1, pos_b0, pos_b1,
          sem_r0, sem_r1, sem_p0, sem_p1):
    wid = lax.axis_index("s") * 2 + lax.axis_index("c")
    b0 = wid * NB_W

    pltpu.sync_copy(x_hbm.at[pl.ds(b0 * 2 * S, NB_W * 2 * S)], x_v)
    pltpu.sync_copy(ppost_hbm, ppost_v)
    pltpu.sync_copy(ppara_hbm, ppara_v)
    pltpu.sync_copy(dpost_hbm, dpost_v)
    pltpu.sync_copy(dpara_hbm, dpara_v)

    iota = lax.iota(jnp.int32, LANES)
    rel_bufs = (rel_b0, rel_b1)
    rel_sems = (sem_r0, sem_r1)
    pos_bufs = (pos_b0, pos_b1)
    pos_sems = (sem_p0, sem_p1)
    cols = [_splat(c) for c in range(32)]
    jv32s = [(iota + c) * 32 for c in J_CHUNKS]

    def batch_pair(bp, carry):
        for b_par in range(2):
            bl = bp * 2 + b_par
            b = b0 + bl
            xoff = bl * 2 * S
            xj0 = [x_v[pl.ds(xoff + c, LANES)] for c in J_CHUNKS]
            xj1 = [x_v[pl.ds(xoff + S + c, LANES)] for c in J_CHUNKS]

            pos_buf = pos_bufs[b_par]
            psem = pos_sems[b_par]

            @pl.when(bp > 0)
            def _wait_pos():
                pltpu.make_async_copy(pos_buf, pos_hbm.at[pl.ds(b * POS_ROW, POS_ROW)], psem).wait()

            for ci in range(len(J_CHUNKS)):
                i0 = jnp.minimum(jnp.abs(xj0[ci]), 15) * 16
                i1 = jnp.minimum(jnp.abs(xj1[ci]), 15) * 16
                jv32 = jv32s[ci]
                for col in range(16):
                    plsc.store_scatter(pos_buf, [jv32 + col],
                                       plsc.load_gather(ppost_v, [i0 + col]))
                    plsc.store_scatter(pos_buf, [jv32 + (col + 16)],
                                       plsc.load_gather(ppara_v, [i1 + col]))
            pltpu.make_async_copy(pos_buf, pos_hbm.at[pl.ds(b * POS_ROW, POS_ROW)], psem).start()

            for h in range(2):
                rel_buf = rel_bufs[h]
                rsem = rel_sems[h]
                dst = rel_hbm.at[pl.ds(b * 2 * REL_HB + h * REL_HB, REL_HB)]

                @pl.when(bl > 0)
                def _wait_rel():
                    pltpu.make_async_copy(rel_buf, dst, rsem).wait()

                def i_row(il, c2):
                    ig = h * HALF + il
                    xi0 = plsc.load_gather(x_v, [_splat(xoff + ig)])
                    xi1 = plsc.load_gather(x_v, [_splat(xoff + S + ig)])
                    obase = il * REL_ROW
                    for ci in range(len(J_CHUNKS)):
                        a = jnp.minimum(jnp.abs(xj0[ci] - xi0), 15) * 16
                        p = jnp.minimum(jnp.abs(xj1[ci] - xi1), 3) * 16
                        jv32 = jv32s[ci] + obase
                        for col in range(16):
                            plsc.store_scatter(
                                rel_buf, [jv32 + col],
                                plsc.load_gather(dpara_v, [a + col]))
                            plsc.store_scatter(
                                rel_buf, [jv32 + (col + 16)],
                                plsc.load_gather(dpost_v, [p + col]))
                    return c2

                lax.fori_loop(0, HALF, i_row, 0)
                pltpu.make_async_copy(rel_buf, dst, rsem).start()
        return carry

    lax.fori_loop(0, NB_W // 2, batch_pair, 0)

    b_last = b0 + NB_W - 1
    for h in range(2):
        pltpu.make_async_copy(
            rel_bufs[h],
            rel_hbm.at[pl.ds(b_last * 2 * REL_HB + h * REL_HB, REL_HB)],
            rel_sems[h]).wait()
    for b_par in range(2):
        pltpu.make_async_copy(pos_bufs[b_par],
                              pos_hbm.at[pl.ds(b_last * POS_ROW, POS_ROW)],
                              pos_sems[b_par]).wait()


@jax.jit
def _sc_position_layer(x, ppost, ppara, dpost, dpara):
    mesh = plsc.VectorSubcoreMesh(core_axis_name="c", subcore_axis_name="s")
    f = pl.kernel(
        _body,
        out_type=(jax.ShapeDtypeStruct((B * POS_ROW,), jnp.float32),
                  jax.ShapeDtypeStruct((B * 2 * REL_HB,), jnp.float32)),
        mesh=mesh,
        scratch_types=[
            pltpu.VMEM((NB_W * 2 * S,), jnp.int32),
            pltpu.VMEM((256,), jnp.float32),
            pltpu.VMEM((256,), jnp.float32),
            pltpu.VMEM((64,), jnp.float32),
            pltpu.VMEM((256,), jnp.float32),
            pltpu.VMEM((REL_HB,), jnp.float32),
            pltpu.VMEM((REL_HB,), jnp.float32),
            pltpu.VMEM((POS_ROW,), jnp.float32),
            pltpu.VMEM((POS_ROW,), jnp.float32),
            pltpu.SemaphoreType.DMA,
            pltpu.SemaphoreType.DMA,
            pltpu.SemaphoreType.DMA,
            pltpu.SemaphoreType.DMA,
        ],
        compiler_params=pltpu.CompilerParams(needs_layout_passes=False),
    )
    return f(x, ppost, ppara, dpost, dpara)


def kernel(x_position_info, pos_post_emb, pos_para_emb, dist_post_emb, dist_para_emb):
    x = x_position_info.astype(jnp.int32).transpose(0, 2, 1).reshape(B * 2 * S)
    pos_flat, rel_flat = _sc_position_layer(
        x, pos_post_emb.reshape(-1), pos_para_emb.reshape(-1),
        dist_post_emb.reshape(-1), dist_para_emb.reshape(-1))
    return (pos_flat.reshape(B, S, 32), rel_flat.reshape(B, S, S, 32))
